# baseline (speedup 1.0000x reference)
import functools

import jax
import jax.numpy as jnp
from jax import lax
from jax.experimental import pallas as pl
from jax.experimental.pallas import tpu as pltpu
from jax.experimental.pallas import tpu_sc as plsc

N, E, D, DE, DOUT = 10000, 320000, 128, 16, 128

NC, NS, L = 2, 16, 16
NW = NC * NS
EW = E // NW
C = 80
NCHUNK = EW // C
NTRIPLE = (NCHUNK - 2) // 3
VPR = DOUT // L


def _node_mm_body(x_ref, w1_ref, w2_ref, y1_ref, y2_ref):
    xv = x_ref[...]
    y1_ref[...] = jnp.dot(xv, w1_ref[...], preferred_element_type=jnp.float32)
    y2_ref[...] = jnp.dot(xv, w2_ref[...], preferred_element_type=jnp.float32)


_node_mm = pl.pallas_call(
    _node_mm_body,
    out_shape=[
        jax.ShapeDtypeStruct((N, DOUT), jnp.float32),
        jax.ShapeDtypeStruct((N, DOUT), jnp.float32),
    ],
)


def _ea_mm_body(eat_ref, w3_ref, b_ref, o_ref):
    o_ref[...] = (
        jax.lax.dot_general(
            eat_ref[...], w3_ref[...],
            (((0,), (0,)), ((), ())),
            preferred_element_type=jnp.float32,
        )
        + b_ref[...]
    )


_EB = 12800

_ea_mm = pl.pallas_call(
    _ea_mm_body,
    grid=(E // _EB,),
    in_specs=[
        pl.BlockSpec((DE, _EB), lambda i: (0, i)),
        pl.BlockSpec((DE, DOUT), lambda i: (0, 0)),
        pl.BlockSpec((1, DOUT), lambda i: (0, 0)),
    ],
    out_specs=pl.BlockSpec((_EB, DOUT), lambda i: (i, 0)),
    out_shape=jax.ShapeDtypeStruct((E, DOUT), jnp.float32),
)


def _sc_gather_body(y1_hbm, y2_hbm, ea_hbm, row_hbm, col_hbm, out_hbm,
                    row_v, col_v,
                    a1, a2, ao, b1, b2, bo, c1, c2, co,
                    sga, sgb, sgc, swa, swb, swc):
    wid = lax.axis_index("s") * NC + lax.axis_index("c")
    base = wid * EW
    pltpu.sync_copy(row_hbm.at[pl.ds(base, EW)], row_v)
    pltpu.sync_copy(col_hbm.at[pl.ds(base, EW)], col_v)

    slots = ((a1, a2, ao, sga, swa),
             (b1, b2, bo, sgb, swb),
             (c1, c2, co, sgc, swc))

    def issue_gathers(c, slot):
        g1, g2, oe, sg, _ = slot
        off = c * C
        pltpu.async_copy(y1_hbm.at[row_v.at[pl.ds(off, C)]], g1, sg)
        pltpu.async_copy(y2_hbm.at[col_v.at[pl.ds(off, C)]], g2, sg)
        pltpu.async_copy(ea_hbm.at[pl.ds(base + off, C)], oe, sg)

    def wait_gathers(c, slot):
        g1, g2, oe, sg, _ = slot
        off = c * C
        pltpu.make_async_copy(y1_hbm.at[row_v.at[pl.ds(off, C)]], g1, sg).wait()
        pltpu.make_async_copy(y2_hbm.at[col_v.at[pl.ds(off, C)]], g2, sg).wait()
        pltpu.make_async_copy(ea_hbm.at[pl.ds(base + off, C)], oe, sg).wait()

    def issue_write(c, slot):
        oe, sw = slot[2], slot[4]
        pltpu.async_copy(oe, out_hbm.at[pl.ds(base + c * C, C)], sw)

    def wait_write(c, slot):
        oe, sw = slot[2], slot[4]
        pltpu.make_async_copy(oe, out_hbm.at[pl.ds(base + c * C, C)], sw).wait()

    def compute(slot):
        g1, g2, oe = slot[0], slot[1], slot[2]

        def row_body(i, carry):
            for k in range(VPR):
                sl = pl.ds(k * L, L)
                plsc.addupdate(oe.at[i, sl], g1[i, sl] + g2[i, sl])
            return carry

        lax.fori_loop(0, C, row_body, 0)

    def step(c, slot):
        wait_gathers(c, slot)
        compute(slot)
        issue_write(c, slot)

    def refill(cn, slot):
        wait_write(cn - 3, slot)
        issue_gathers(cn, slot)

    for s in range(3):
        issue_gathers(s, slots[s])

    def triple_body(p, carry):
        ca = 3 * p
        step(ca, slots[0])
        step(ca + 1, slots[1])
        refill(ca + 3, slots[0])
        step(ca + 2, slots[2])
        refill(ca + 4, slots[1])

        @pl.when(ca + 5 < NCHUNK)
        def _():
            refill(ca + 5, slots[2])

        return carry

    lax.fori_loop(0, NTRIPLE, triple_body, 0)

    step(NCHUNK - 2, slots[0])
    step(NCHUNK - 1, slots[1])
    wait_write(NCHUNK - 2, slots[0])
    wait_write(NCHUNK - 1, slots[1])
    wait_write(NCHUNK - 3, slots[2])


_sc_gather = functools.partial(
    pl.kernel,
    out_type=jax.ShapeDtypeStruct((E, DOUT), jnp.float32),
    mesh=plsc.VectorSubcoreMesh(core_axis_name="c", subcore_axis_name="s"),
    scratch_types=[
        pltpu.VMEM((EW,), jnp.int32),
        pltpu.VMEM((EW,), jnp.int32),
        pltpu.VMEM((C, DOUT), jnp.float32),
        pltpu.VMEM((C, DOUT), jnp.float32),
        pltpu.VMEM((C, DOUT), jnp.float32),
        pltpu.VMEM((C, DOUT), jnp.float32),
        pltpu.VMEM((C, DOUT), jnp.float32),
        pltpu.VMEM((C, DOUT), jnp.float32),
        pltpu.VMEM((C, DOUT), jnp.float32),
        pltpu.VMEM((C, DOUT), jnp.float32),
        pltpu.VMEM((C, DOUT), jnp.float32),
        pltpu.SemaphoreType.DMA,
        pltpu.SemaphoreType.DMA,
        pltpu.SemaphoreType.DMA,
        pltpu.SemaphoreType.DMA,
        pltpu.SemaphoreType.DMA,
        pltpu.SemaphoreType.DMA,
    ],
)(_sc_gather_body)


def kernel(x, edge_index, edge_attr, W, b):
    w1 = W[:D]
    w2 = W[D:2 * D]
    w3 = W[2 * D:]
    row = edge_index[0]
    col = edge_index[1]
    y1, y2 = _node_mm(x, w1, w2)
    ea = _ea_mm(edge_attr.T, w3, b.reshape(1, DOUT))
    return _sc_gather(y1, y2, ea, row, col)


# speedup vs baseline: 1.0010x; 1.0010x over previous
import functools

import jax
import jax.numpy as jnp
from jax import lax
from jax.experimental import pallas as pl
from jax.experimental.pallas import tpu as pltpu
from jax.experimental.pallas import tpu_sc as plsc

N, E, D, DE, DOUT = 10000, 320000, 128, 16, 128

NC, NS, L = 2, 16, 16
NW = NC * NS
EW = E // NW
C = 80
NCHUNK = EW // C
NTRIPLE = (NCHUNK - 2) // 3
VPR = DOUT // L


def _node_mm_body(x_ref, w1_ref, w2_ref, y1_ref, y2_ref):
    xv = x_ref[...]
    y1_ref[...] = jnp.dot(xv, w1_ref[...], preferred_element_type=jnp.float32)
    y2_ref[...] = jnp.dot(xv, w2_ref[...], preferred_element_type=jnp.float32)


_node_mm = pl.pallas_call(
    _node_mm_body,
    out_shape=[
        jax.ShapeDtypeStruct((N, DOUT), jnp.float32),
        jax.ShapeDtypeStruct((N, DOUT), jnp.float32),
    ],
)


def _ea_mm_body(eat_ref, w3_ref, b_ref, o_ref):
    o_ref[...] = (
        jax.lax.dot_general(
            eat_ref[...], w3_ref[...],
            (((0,), (0,)), ((), ())),
            preferred_element_type=jnp.float32,
        )
        + b_ref[...]
    )


_EB = 12800

_ea_mm = pl.pallas_call(
    _ea_mm_body,
    grid=(E // _EB,),
    in_specs=[
        pl.BlockSpec((DE, _EB), lambda i: (0, i)),
        pl.BlockSpec((DE, DOUT), lambda i: (0, 0)),
        pl.BlockSpec((1, DOUT), lambda i: (0, 0)),
    ],
    out_specs=pl.BlockSpec((_EB, DOUT), lambda i: (i, 0)),
    out_shape=jax.ShapeDtypeStruct((E, DOUT), jnp.float32),
)


def _sc_gather_body(y1_hbm, y2_hbm, ea_hbm, row_hbm, col_hbm, out_hbm,
                    row_v, col_v,
                    a1, a2, ao, b1, b2, bo, c1, c2, co,
                    sga, sgb, sgc, swa, swb, swc):
    wid = lax.axis_index("s") * NC + lax.axis_index("c")
    base = wid * EW
    pltpu.sync_copy(row_hbm.at[pl.ds(base, EW)], row_v)
    pltpu.sync_copy(col_hbm.at[pl.ds(base, EW)], col_v)

    slots = ((a1, a2, ao, sga, swa),
             (b1, b2, bo, sgb, swb),
             (c1, c2, co, sgc, swc))

    def issue_gathers(c, slot):
        g1, g2, oe, sg, _ = slot
        off = c * C
        pltpu.async_copy(y1_hbm.at[row_v.at[pl.ds(off, C)]], g1, sg)
        pltpu.async_copy(y2_hbm.at[col_v.at[pl.ds(off, C)]], g2, sg)
        pltpu.async_copy(ea_hbm.at[pl.ds(base + off, C)], oe, sg)

    def wait_gathers(c, slot):
        g1, g2, oe, sg, _ = slot
        off = c * C
        pltpu.make_async_copy(y1_hbm.at[row_v.at[pl.ds(off, C)]], g1, sg).wait()
        pltpu.make_async_copy(y2_hbm.at[col_v.at[pl.ds(off, C)]], g2, sg).wait()
        pltpu.make_async_copy(ea_hbm.at[pl.ds(base + off, C)], oe, sg).wait()

    def issue_write(c, slot):
        oe, sw = slot[2], slot[4]
        pltpu.async_copy(oe, out_hbm.at[pl.ds(base + c * C, C)], sw)

    def wait_write(c, slot):
        oe, sw = slot[2], slot[4]
        pltpu.make_async_copy(oe, out_hbm.at[pl.ds(base + c * C, C)], sw).wait()

    def compute(slot):
        g1, g2, oe = slot[0], slot[1], slot[2]

        @plsc.parallel_loop(0, C, unroll=2)
        def _(i):
            for k in range(VPR):
                sl = pl.ds(k * L, L)
                oe[i, sl] = oe[i, sl] + g1[i, sl] + g2[i, sl]

    def step(c, slot):
        wait_gathers(c, slot)
        compute(slot)
        issue_write(c, slot)

    def refill(cn, slot):
        wait_write(cn - 3, slot)
        issue_gathers(cn, slot)

    for s in range(3):
        issue_gathers(s, slots[s])

    def triple_body(p, carry):
        ca = 3 * p
        step(ca, slots[0])
        step(ca + 1, slots[1])
        refill(ca + 3, slots[0])
        step(ca + 2, slots[2])
        refill(ca + 4, slots[1])

        @pl.when(ca + 5 < NCHUNK)
        def _():
            refill(ca + 5, slots[2])

        return carry

    lax.fori_loop(0, NTRIPLE, triple_body, 0)

    step(NCHUNK - 2, slots[0])
    step(NCHUNK - 1, slots[1])
    wait_write(NCHUNK - 2, slots[0])
    wait_write(NCHUNK - 1, slots[1])
    wait_write(NCHUNK - 3, slots[2])


_sc_gather = functools.partial(
    pl.kernel,
    out_type=jax.ShapeDtypeStruct((E, DOUT), jnp.float32),
    mesh=plsc.VectorSubcoreMesh(core_axis_name="c", subcore_axis_name="s"),
    scratch_types=[
        pltpu.VMEM((EW,), jnp.int32),
        pltpu.VMEM((EW,), jnp.int32),
        pltpu.VMEM((C, DOUT), jnp.float32),
        pltpu.VMEM((C, DOUT), jnp.float32),
        pltpu.VMEM((C, DOUT), jnp.float32),
        pltpu.VMEM((C, DOUT), jnp.float32),
        pltpu.VMEM((C, DOUT), jnp.float32),
        pltpu.VMEM((C, DOUT), jnp.float32),
        pltpu.VMEM((C, DOUT), jnp.float32),
        pltpu.VMEM((C, DOUT), jnp.float32),
        pltpu.VMEM((C, DOUT), jnp.float32),
        pltpu.SemaphoreType.DMA,
        pltpu.SemaphoreType.DMA,
        pltpu.SemaphoreType.DMA,
        pltpu.SemaphoreType.DMA,
        pltpu.SemaphoreType.DMA,
        pltpu.SemaphoreType.DMA,
    ],
)(_sc_gather_body)


def kernel(x, edge_index, edge_attr, W, b):
    w1 = W[:D]
    w2 = W[D:2 * D]
    w3 = W[2 * D:]
    row = edge_index[0]
    col = edge_index[1]
    y1, y2 = _node_mm(x, w1, w2)
    ea = _ea_mm(edge_attr.T, w3, b.reshape(1, DOUT))
    return _sc_gather(y1, y2, ea, row, col)


# speedup vs baseline: 1.0737x; 1.0725x over previous
import functools

import jax
import jax.numpy as jnp
from jax import lax
from jax.experimental import pallas as pl
from jax.experimental.pallas import tpu as pltpu
from jax.experimental.pallas import tpu_sc as plsc

N, E, D, DE, DOUT = 10000, 320000, 128, 16, 128

NC, NS, L = 2, 16, 16
NW = NC * NS
EW = E // NW
C = 80
NCHUNK = EW // C
NTRIPLE = (NCHUNK - 2) // 3
VPR = DOUT // L


def _node_mm_body(x_ref, w1_ref, w2_ref, y1_ref, y2_ref):
    xv = x_ref[...]
    y1_ref[...] = jnp.dot(xv, w1_ref[...], preferred_element_type=jnp.float32)
    y2_ref[...] = jnp.dot(xv, w2_ref[...], preferred_element_type=jnp.float32)


_node_mm = pl.pallas_call(
    _node_mm_body,
    out_shape=[
        jax.ShapeDtypeStruct((N, DOUT), jnp.float32),
        jax.ShapeDtypeStruct((N, DOUT), jnp.float32),
    ],
)


def _sc_gather_body(y1_hbm, y2_hbm, row_hbm, col_hbm, out_hbm,
                    row_v, col_v,
                    a1, a2, ao, b1, b2, bo, c1, c2, co,
                    sga, sgb, sgc, swa, swb, swc):
    wid = lax.axis_index("s") * NC + lax.axis_index("c")
    base = wid * EW
    pltpu.sync_copy(row_hbm.at[pl.ds(base, EW)], row_v)
    pltpu.sync_copy(col_hbm.at[pl.ds(base, EW)], col_v)

    slots = ((a1, a2, ao, sga, swa),
             (b1, b2, bo, sgb, swb),
             (c1, c2, co, sgc, swc))

    def issue_gathers(c, slot):
        g1, g2, _, sg, _ = slot
        off = c * C
        pltpu.async_copy(y1_hbm.at[row_v.at[pl.ds(off, C)]], g1, sg)
        pltpu.async_copy(y2_hbm.at[col_v.at[pl.ds(off, C)]], g2, sg)

    def wait_gathers(c, slot):
        g1, g2, _, sg, _ = slot
        off = c * C
        pltpu.make_async_copy(y1_hbm.at[row_v.at[pl.ds(off, C)]], g1, sg).wait()
        pltpu.make_async_copy(y2_hbm.at[col_v.at[pl.ds(off, C)]], g2, sg).wait()

    def issue_write(c, slot):
        o, sw = slot[2], slot[4]
        pltpu.async_copy(o, out_hbm.at[pl.ds(base + c * C, C)], sw)

    def wait_write(c, slot):
        o, sw = slot[2], slot[4]
        pltpu.make_async_copy(o, out_hbm.at[pl.ds(base + c * C, C)], sw).wait()

    def compute(slot):
        g1, g2, o = slot[0], slot[1], slot[2]

        @plsc.parallel_loop(0, C, unroll=2)
        def _(i):
            for k in range(VPR):
                sl = pl.ds(k * L, L)
                o[i, sl] = g1[i, sl] + g2[i, sl]

    def step(c, slot):
        wait_gathers(c, slot)
        @pl.when(c >= 3)
        def _():
            wait_write(c - 3, slot)

        compute(slot)
        issue_write(c, slot)
        issue_gathers(c + 3, slot)

    def step_noissue(c, slot):
        wait_gathers(c, slot)

        @pl.when(c >= 3)
        def _():
            wait_write(c - 3, slot)

        compute(slot)
        issue_write(c, slot)

    for s in range(3):
        issue_gathers(s, slots[s])

    def triple_body(p, carry):
        ca = 3 * p
        step(ca, slots[0])
        step(ca + 1, slots[1])

        @pl.when(ca + 5 < NCHUNK)
        def _():
            step(ca + 2, slots[2])

        @pl.when(ca + 5 >= NCHUNK)
        def _():
            step_noissue(ca + 2, slots[2])

        return carry

    lax.fori_loop(0, NTRIPLE, triple_body, 0)

    step_noissue(NCHUNK - 2, slots[0])
    step_noissue(NCHUNK - 1, slots[1])
    wait_write(NCHUNK - 3, slots[2])
    wait_write(NCHUNK - 2, slots[0])
    wait_write(NCHUNK - 1, slots[1])


_sc_gather = functools.partial(
    pl.kernel,
    out_type=jax.ShapeDtypeStruct((E, DOUT), jnp.float32),
    mesh=plsc.VectorSubcoreMesh(core_axis_name="c", subcore_axis_name="s"),
    scratch_types=[
        pltpu.VMEM((EW,), jnp.int32),
        pltpu.VMEM((EW,), jnp.int32),
        pltpu.VMEM((C, DOUT), jnp.float32),
        pltpu.VMEM((C, DOUT), jnp.float32),
        pltpu.VMEM((C, DOUT), jnp.float32),
        pltpu.VMEM((C, DOUT), jnp.float32),
        pltpu.VMEM((C, DOUT), jnp.float32),
        pltpu.VMEM((C, DOUT), jnp.float32),
        pltpu.VMEM((C, DOUT), jnp.float32),
        pltpu.VMEM((C, DOUT), jnp.float32),
        pltpu.VMEM((C, DOUT), jnp.float32),
        pltpu.SemaphoreType.DMA,
        pltpu.SemaphoreType.DMA,
        pltpu.SemaphoreType.DMA,
        pltpu.SemaphoreType.DMA,
        pltpu.SemaphoreType.DMA,
        pltpu.SemaphoreType.DMA,
    ],
)(_sc_gather_body)


def _edge_out_body(s_ref, eat_ref, w3_ref, b_ref, o_ref):
    o_ref[...] = (
        s_ref[...]
        + jax.lax.dot_general(
            eat_ref[...], w3_ref[...],
            (((0,), (0,)), ((), ())),
            preferred_element_type=jnp.float32,
        )
        + b_ref[...]
    )


_EB = 12800

_edge_out = pl.pallas_call(
    _edge_out_body,
    grid=(E // _EB,),
    in_specs=[
        pl.BlockSpec((_EB, DOUT), lambda i: (i, 0)),
        pl.BlockSpec((DE, _EB), lambda i: (0, i)),
        pl.BlockSpec((DE, DOUT), lambda i: (0, 0)),
        pl.BlockSpec((1, DOUT), lambda i: (0, 0)),
    ],
    out_specs=pl.BlockSpec((_EB, DOUT), lambda i: (i, 0)),
    out_shape=jax.ShapeDtypeStruct((E, DOUT), jnp.float32),
)


def kernel(x, edge_index, edge_attr, W, b):
    w1 = W[:D]
    w2 = W[D:2 * D]
    w3 = W[2 * D:]
    row = edge_index[0]
    col = edge_index[1]
    y1, y2 = _node_mm(x, w1, w2)
    s = _sc_gather(y1, y2, row, col)
    return _edge_out(s, edge_attr.T, w3, b.reshape(1, DOUT))
